# baseline (speedup 1.0000x reference)
import functools

import jax
import jax.numpy as jnp
from jax import lax
from jax.experimental import pallas as pl
from jax.experimental.pallas import tpu as pltpu
from jax.experimental.pallas import tpu_sc as plsc

_NC = 2
_NS = 16
_NW = _NC * _NS
_C = 128
_NBUF = 4
_PRIME = 2
_L = 16


@jax.jit
def _gather_sc(tab32, idx_flat):
    n = idx_flat.shape[0]
    dw = tab32.shape[1]
    d = 2 * dw
    b_per_w = n // _NW
    nchunks = b_per_w // _C
    ngroups = nchunks // _NBUF
    assert b_per_w * _NW == n
    assert nchunks * _C == b_per_w
    assert ngroups * _NBUF == nchunks
    assert nchunks > _NBUF >= _PRIME

    mesh = plsc.VectorSubcoreMesh(core_axis_name="c", subcore_axis_name="s")

    @functools.partial(
        pl.kernel,
        out_type=jax.ShapeDtypeStruct((n, d), jnp.float32),
        mesh=mesh,
        scratch_types=[
            pltpu.VMEM((b_per_w,), jnp.int32),
            pltpu.VMEM((_NBUF, _C, dw), jnp.int32),
            pltpu.VMEM((_NBUF, _C, d), jnp.float32),
        ]
        + [pltpu.SemaphoreType.DMA] * (2 * _NBUF),
        compiler_params=pltpu.CompilerParams(use_tc_tiling_on_sc=False),
    )
    def k(tab_hbm, idx_hbm, out_hbm, idx_v, rows16_v, out32_v, *sems):
        gsem = sems[:_NBUF]
        wsem = sems[_NBUF:]
        wid = lax.axis_index("s") * _NC + lax.axis_index("c")
        base = wid * b_per_w
        pltpu.sync_copy(idx_hbm.at[pl.ds(base, b_per_w)], idx_v)

        def gather(g, b):
            pltpu.async_copy(
                tab_hbm.at[idx_v.at[pl.ds(g * _C, _C)]], rows16_v.at[b], gsem[b]
            )

        def wait_gather(b):
            pltpu.make_async_copy(
                tab_hbm.at[idx_v.at[pl.ds(0, _C)]], rows16_v.at[b], gsem[b]
            ).wait()

        def write(g, b):
            pltpu.async_copy(
                out32_v.at[b], out_hbm.at[pl.ds(base + g * _C, _C)], wsem[b]
            )

        def wait_write(b):
            pltpu.make_async_copy(
                out32_v.at[b], out_hbm.at[pl.ds(base, _C)], wsem[b]
            ).wait()

        def convert(b):
            @pl.loop(0, _C)
            def _(r):
                for wg in range(dw // _L):
                    x = rows16_v[b, r, pl.ds(_L * wg, _L)]
                    lo = lax.bitcast_convert_type(x << 16, jnp.float32)
                    hi = lax.bitcast_convert_type(
                        x & jnp.int32(-65536), jnp.float32
                    )
                    out32_v[b, r, pl.ds(2 * _L * wg, _L)] = lo
                    out32_v[b, r, pl.ds(2 * _L * wg + _L, _L)] = hi

        for b in range(_PRIME):
            gather(b, b)

        @pl.loop(0, ngroups)
        def _(g0):
            for b in range(_NBUF):
                g = g0 * _NBUF + b
                wait_gather(b)

                @pl.when(g >= _NBUF)
                def _w():
                    wait_write(b)

                convert(b)
                write(g, b)
                gf = g + _PRIME

                @pl.when(gf < nchunks)
                def _issue():
                    gather(gf, (b + _PRIME) % _NBUF)

        for b in range(_NBUF):
            wait_write(b)

    return k(tab32, idx_flat)


def kernel(locus_indices, table):
    b, s = locus_indices.shape
    v, d = table.shape
    idx_flat = locus_indices.reshape(b * s).astype(jnp.int32)
    tabp = table.astype(jnp.bfloat16).reshape(v, d // 32, 2, _L)
    tabp = tabp.transpose(0, 1, 3, 2)
    tab32 = jax.lax.bitcast_convert_type(tabp.reshape(v, d // 2, 2), jnp.int32)
    out = _gather_sc(tab32, idx_flat)
    return out.reshape(b, s, d)


# speedup vs baseline: 2.0996x; 2.0996x over previous
import functools

import jax
import jax.numpy as jnp
from jax import lax
from jax.experimental import pallas as pl
from jax.experimental.pallas import tpu as pltpu
from jax.experimental.pallas import tpu_sc as plsc

_NC = 2
_NS = 16
_NW = _NC * _NS
_C = 128
_NBUF = 5
_PRIME = 4


@jax.jit
def _gather_sc(table, idx_flat):
    n = idx_flat.shape[0]
    d = table.shape[1]
    b_per_w = n // _NW
    nchunks = b_per_w // _C
    ngroups = nchunks // _NBUF
    assert b_per_w * _NW == n
    assert nchunks * _C == b_per_w
    assert ngroups * _NBUF == nchunks
    assert nchunks > _NBUF

    mesh = plsc.VectorSubcoreMesh(core_axis_name="c", subcore_axis_name="s")

    @functools.partial(
        pl.kernel,
        out_type=jax.ShapeDtypeStruct((n, d), table.dtype),
        mesh=mesh,
        scratch_types=[
            pltpu.VMEM((b_per_w,), jnp.int32),
            pltpu.VMEM((_NBUF, _C, d), jnp.float32),
        ]
        + [pltpu.SemaphoreType.DMA] * (2 * _NBUF),
    )
    def k(table_hbm, idx_hbm, out_hbm, idx_v, rows_v, *sems):
        gsem = sems[:_NBUF]
        wsem = sems[_NBUF:]
        wid = lax.axis_index("s") * _NC + lax.axis_index("c")
        base = wid * b_per_w
        pltpu.sync_copy(idx_hbm.at[pl.ds(base, b_per_w)], idx_v)

        def gather(g, b):
            pltpu.async_copy(
                table_hbm.at[idx_v.at[pl.ds(g * _C, _C)]], rows_v.at[b], gsem[b]
            )

        def wait_gather(b):
            pltpu.make_async_copy(
                table_hbm.at[idx_v.at[pl.ds(0, _C)]], rows_v.at[b], gsem[b]
            ).wait()

        def write(g, b):
            pltpu.async_copy(
                rows_v.at[b], out_hbm.at[pl.ds(base + g * _C, _C)], wsem[b]
            )

        def wait_write(b):
            pltpu.make_async_copy(
                rows_v.at[b], out_hbm.at[pl.ds(base, _C)], wsem[b]
            ).wait()

        for b in range(_PRIME):
            gather(b, b)

        @pl.loop(0, ngroups)
        def _(g0):
            for b in range(_NBUF):
                g = g0 * _NBUF + b
                wait_gather(b)
                write(g, b)
                gf = g + _PRIME
                bf = (b + _PRIME) % _NBUF

                @pl.when(gf < nchunks)
                def _issue():
                    @pl.when(g >= _NBUF - _PRIME)
                    def _w():
                        wait_write(bf)

                    gather(gf, bf)

        for b in range(_NBUF):
            wait_write(b)

    return k(table, idx_flat)


def kernel(locus_indices, table):
    b, s = locus_indices.shape
    idx_flat = locus_indices.reshape(b * s).astype(jnp.int32)
    out = _gather_sc(table, idx_flat)
    return out.reshape(b, s, table.shape[1])
